# baseline (speedup 1.0000x reference)
import functools

import jax
import jax.numpy as jnp
from jax import lax
from jax.experimental import pallas as pl
from jax.experimental.pallas import tpu as pltpu
from jax.experimental.pallas import tpu_sc as plsc

L = 16


@functools.partial(jax.jit, static_argnames=())
def kernel(ids, table):
    B, H = ids.shape
    V, E = table.shape
    info = plsc.get_sparse_core_info()
    NC, NS = info.num_cores, info.num_subcores
    NW = NC * NS
    BPW = B // NW
    RPG = 2
    IPG = RPG * H
    GPW = BPW // RPG

    ids_r = ids.astype(jnp.int32).reshape(NW, GPW, IPG)

    mesh = plsc.VectorSubcoreMesh(core_axis_name="c", subcore_axis_name="s")

    @functools.partial(
        pl.kernel,
        out_type=jax.ShapeDtypeStruct((NW, BPW * E), jnp.float32),
        mesh=mesh,
        scratch_types=[
            pltpu.VMEM((GPW, IPG), jnp.int32),
            pltpu.VMEM((IPG, E), jnp.float32),
            pltpu.VMEM((BPW * E,), jnp.float32),
            pltpu.SemaphoreType.DMA,
        ],
        compiler_params=pltpu.CompilerParams(use_tc_tiling_on_sc=False),
    )
    def sc_kernel(ids_hbm, table_hbm, out_hbm, idx_v, rows_v, out_v, sem):
        wid = lax.axis_index("s") * NC + lax.axis_index("c")
        pltpu.sync_copy(ids_hbm.at[wid], idx_v)
        inv = jnp.full((L,), 1.0 / H, dtype=jnp.float32)

        def body(g, carry):
            cp = pltpu.make_async_copy(
                table_hbm.at[idx_v.at[g]], rows_v, sem)
            cp.start()
            cp.wait()
            a0 = rows_v[0, 0:L]
            a1 = rows_v[0, L:2 * L]
            b0 = rows_v[H, 0:L]
            b1 = rows_v[H, L:2 * L]
            for h in range(1, H):
                a0 = a0 + rows_v[h, 0:L]
                a1 = a1 + rows_v[h, L:2 * L]
                b0 = b0 + rows_v[H + h, 0:L]
                b1 = b1 + rows_v[H + h, L:2 * L]
            base = g * (RPG * E)
            out_v[pl.ds(base, L)] = a0 * inv
            out_v[pl.ds(base + L, L)] = a1 * inv
            out_v[pl.ds(base + 2 * L, L)] = b0 * inv
            out_v[pl.ds(base + 3 * L, L)] = b1 * inv
            return carry

        lax.fori_loop(0, GPW, body, 0)
        pltpu.sync_copy(out_v, out_hbm.at[wid])

    out = sc_kernel(ids_r, table)
    return out.reshape(B, E)
